# baseline (speedup 1.0000x reference)
import jax
import jax.numpy as jnp
from jax.experimental import pallas as pl


def _add_body(x_ref, p_ref, o_ref):
    o_ref[...] = x_ref[...] + p_ref[...]


def kernel(x, pos_emb):
    B, S, D = x.shape
    SD = S * D
    x2 = x.reshape(B, SD)
    p2 = pos_emb.reshape(1, SD)
    BB = 128
    out = pl.pallas_call(
        _add_body,
        grid=(B // BB,),
        in_specs=[
            pl.BlockSpec((BB, SD), lambda i: (i, 0)),
            pl.BlockSpec((1, SD), lambda i: (0, 0)),
        ],
        out_specs=pl.BlockSpec((BB, SD), lambda i: (i, 0)),
        out_shape=jax.ShapeDtypeStruct((B, SD), x.dtype),
    )(x2, p2)
    return out.reshape(B, S, D)
